# baseline (speedup 1.0000x reference)
import functools

import jax
import jax.numpy as jnp
from jax import lax
from jax.experimental import pallas as pl
from jax.experimental.pallas import tpu as pltpu
from jax.experimental.pallas import tpu_sc as plsc

HIDDEN = 1024
B, S = 4, 8192
N = B * S
NC, NS = 2, 16
NW = NC * NS
TPW = N // NW
C = 16
NCH = TPW // C
LANES = 16
HPACK = HIDDEN // 2
NBUF = 2
MASKHI = -65536


def _emb_body(idx1_hbm, idx2_hbm, w1_hbm, w2_hbm, out_hbm,
              idx1_v, idx2_v, bufs_a, bufs_b, outb,
              sem_g0, sem_g1, sem_s0, sem_s1):
    wid = lax.axis_index("s") * NC + lax.axis_index("c")
    base = wid * TPW
    pltpu.sync_copy(idx1_hbm.at[wid], idx1_v)
    pltpu.sync_copy(idx2_hbm.at[wid], idx2_v)
    sem_g = (sem_g0, sem_g1)
    sem_s = (sem_s0, sem_s1)

    def start_gathers(j, slot):
        pltpu.async_copy(w1_hbm.at[idx1_v.at[j]], bufs_a.at[slot], sem_g[slot])
        pltpu.async_copy(w2_hbm.at[idx2_v.at[j]], bufs_b.at[slot], sem_g[slot])

    def wait_gathers(slot):
        pltpu.make_async_copy(w1_hbm.at[idx1_v.at[0]], bufs_a.at[slot],
                              sem_g[slot]).wait()
        pltpu.make_async_copy(w2_hbm.at[idx2_v.at[0]], bufs_b.at[slot],
                              sem_g[slot]).wait()

    def wait_store(slot):
        pltpu.make_async_copy(outb.at[slot],
                              out_hbm.at[pl.ds(base, C)], sem_s[slot]).wait()

    start_gathers(0, 0)

    def outer(jj, carry):
        for b in range(NBUF):
            j = jj * NBUF + b
            nb = 1 - b

            @pl.when(j + 1 < NCH)
            def _():
                start_gathers(j + 1, nb)

            wait_gathers(b)

            @pl.when(j >= NBUF)
            def _():
                wait_store(b)

            @plsc.parallel_loop(0, C, unroll=8)
            def _row(r):
                for g in range(HPACK // LANES):
                    sl = pl.ds(g * LANES, LANES)
                    wa = bufs_a[b, r, sl]
                    wb = bufs_b[b, r, sl]
                    lo = (lax.bitcast_convert_type(wa << 16, jnp.float32)
                          + lax.bitcast_convert_type(wb << 16, jnp.float32))
                    hi = (lax.bitcast_convert_type(wa & MASKHI, jnp.float32)
                          + lax.bitcast_convert_type(wb & MASKHI, jnp.float32))
                    outb[b, r, pl.ds(g * 2 * LANES, LANES)] = lo
                    outb[b, r, pl.ds(g * 2 * LANES + LANES, LANES)] = hi

            pltpu.async_copy(outb.at[b],
                             out_hbm.at[pl.ds(base + j * C, C)], sem_s[b])
        return carry

    lax.fori_loop(0, NCH // NBUF, outer, 0)
    wait_store(0)
    wait_store(1)


_emb = functools.partial(
    pl.kernel,
    mesh=plsc.VectorSubcoreMesh(core_axis_name="c", subcore_axis_name="s"),
    out_type=jax.ShapeDtypeStruct((N, HIDDEN), jnp.float32),
    scratch_types=[
        pltpu.VMEM((NCH, C), jnp.int32),
        pltpu.VMEM((NCH, C), jnp.int32),
        pltpu.VMEM((NBUF, C, HPACK), jnp.int32),
        pltpu.VMEM((NBUF, C, HPACK), jnp.int32),
        pltpu.VMEM((NBUF, C, HIDDEN), jnp.float32),
        pltpu.SemaphoreType.DMA,
        pltpu.SemaphoreType.DMA,
        pltpu.SemaphoreType.DMA,
        pltpu.SemaphoreType.DMA,
    ],
)(_emb_body)


def _pack_table(W):
    v = W.shape[0]
    u = lax.bitcast_convert_type(W.astype(jnp.bfloat16), jnp.uint16)
    u = u.astype(jnp.uint32).reshape(v, HIDDEN // 32, 2, LANES)
    words = u[:, :, 0, :] | (u[:, :, 1, :] << 16)
    return lax.bitcast_convert_type(words, jnp.int32).reshape(v, HPACK)


def kernel(token_type_ids, W1, W2):
    ids = token_type_ids.astype(jnp.int32)
    idx1 = ids[:, :, 1].reshape(NW, NCH, C)
    idx2 = ids[:, :, 2].reshape(NW, NCH, C)
    out = _emb(idx1, idx2, _pack_table(W1), _pack_table(W2))
    return out.reshape(B, S, HIDDEN)


# speedup vs baseline: 1.1034x; 1.1034x over previous
import functools

import jax
import jax.numpy as jnp
from jax import lax
from jax.experimental import pallas as pl
from jax.experimental.pallas import tpu as pltpu
from jax.experimental.pallas import tpu_sc as plsc

HIDDEN = 1024
B, S = 4, 8192
N = B * S
NC, NS = 2, 16
NW = NC * NS
TPW = N // NW
C = 16
NCH = TPW // C
LANES = 16
HPACK = HIDDEN // 2
NBUF = 2
MASKHI = -65536


def _emb_body(idx_hbm, w_hbm, out_hbm, idx_v, bufs, outb,
              sem_g0, sem_g1, sem_s0, sem_s1):
    wid = lax.axis_index("s") * NC + lax.axis_index("c")
    base = wid * TPW
    pltpu.sync_copy(idx_hbm.at[wid], idx_v)
    sem_g = (sem_g0, sem_g1)
    sem_s = (sem_s0, sem_s1)

    def start_gather(j, slot):
        pltpu.async_copy(w_hbm.at[idx_v.at[j]], bufs.at[slot], sem_g[slot])

    def wait_gather(slot):
        pltpu.make_async_copy(w_hbm.at[idx_v.at[0]], bufs.at[slot],
                              sem_g[slot]).wait()

    def wait_store(slot):
        pltpu.make_async_copy(outb.at[slot],
                              out_hbm.at[pl.ds(base, C)], sem_s[slot]).wait()

    start_gather(0, 0)

    def outer(jj, carry):
        for b in range(NBUF):
            j = jj * NBUF + b
            nb = 1 - b

            @pl.when(j + 1 < NCH)
            def _():
                start_gather(j + 1, nb)

            wait_gather(b)

            @pl.when(j >= NBUF)
            def _():
                wait_store(b)

            @plsc.parallel_loop(0, C, unroll=4)
            def _row(r):
                for g in range(HPACK // LANES):
                    sl = pl.ds(g * LANES, LANES)
                    wa = bufs[b, r, sl]
                    wb = bufs[b, r + C, sl]
                    lo = (lax.bitcast_convert_type(wa << 16, jnp.float32)
                          + lax.bitcast_convert_type(wb << 16, jnp.float32))
                    hi = (lax.bitcast_convert_type(wa & MASKHI, jnp.float32)
                          + lax.bitcast_convert_type(wb & MASKHI, jnp.float32))
                    outb[b, r, pl.ds(g * 2 * LANES, LANES)] = lo
                    outb[b, r, pl.ds(g * 2 * LANES + LANES, LANES)] = hi

            pltpu.async_copy(outb.at[b],
                             out_hbm.at[pl.ds(base + j * C, C)], sem_s[b])
        return carry

    lax.fori_loop(0, NCH // NBUF, outer, 0)
    wait_store(0)
    wait_store(1)


_emb = functools.partial(
    pl.kernel,
    mesh=plsc.VectorSubcoreMesh(core_axis_name="c", subcore_axis_name="s"),
    out_type=jax.ShapeDtypeStruct((N, HIDDEN), jnp.float32),
    scratch_types=[
        pltpu.VMEM((NCH, 2 * C), jnp.int32),
        pltpu.VMEM((NBUF, 2 * C, HPACK), jnp.int32),
        pltpu.VMEM((NBUF, C, HIDDEN), jnp.float32),
        pltpu.SemaphoreType.DMA,
        pltpu.SemaphoreType.DMA,
        pltpu.SemaphoreType.DMA,
        pltpu.SemaphoreType.DMA,
    ],
)(_emb_body)


def _pack_table(W):
    v = W.shape[0]
    u = lax.bitcast_convert_type(W.astype(jnp.bfloat16), jnp.uint16)
    u = u.astype(jnp.uint32).reshape(v, HIDDEN // 32, 2, LANES)
    words = u[:, :, 0, :] | (u[:, :, 1, :] << 16)
    return lax.bitcast_convert_type(words, jnp.int32).reshape(v, HPACK)


def kernel(token_type_ids, W1, W2):
    ids = token_type_ids.astype(jnp.int32)
    idx1 = ids[:, :, 1].reshape(NW, NCH, C)
    idx2 = ids[:, :, 2].reshape(NW, NCH, C) + 256
    idxc = jnp.concatenate([idx1, idx2], axis=2)
    wp = jnp.concatenate([_pack_table(W1), _pack_table(W2)], axis=0)
    out = _emb(idxc, wp)
    return out.reshape(B, S, HIDDEN)


# speedup vs baseline: 1.1254x; 1.0199x over previous
import functools

import jax
import jax.numpy as jnp
from jax import lax
from jax.experimental import pallas as pl
from jax.experimental.pallas import tpu as pltpu
from jax.experimental.pallas import tpu_sc as plsc

HIDDEN = 1024
B, S = 4, 8192
N = B * S
NC, NS = 2, 16
NW = NC * NS
TPW = N // NW
C = 16
NCH = TPW // C
LANES = 16
HPACK = HIDDEN // 2
NBUF = 2
MASKHI = -65536


NGBUF = 4


def _emb_body(idx_hbm, w_hbm, out_hbm, idx_v, bufs, outb,
              sem_g0, sem_g1, sem_g2, sem_g3, sem_s0, sem_s1):
    wid = lax.axis_index("s") * NC + lax.axis_index("c")
    base = wid * TPW
    pltpu.sync_copy(idx_hbm.at[wid], idx_v)
    sem_g = (sem_g0, sem_g1, sem_g2, sem_g3)
    sem_s = (sem_s0, sem_s1)

    def start_gather(j, slot):
        pltpu.async_copy(w_hbm.at[idx_v.at[j]], bufs.at[slot], sem_g[slot])

    def wait_gather(slot):
        pltpu.make_async_copy(w_hbm.at[idx_v.at[0]], bufs.at[slot],
                              sem_g[slot]).wait()

    def wait_store(slot):
        pltpu.make_async_copy(outb.at[slot],
                              out_hbm.at[pl.ds(base, C)], sem_s[slot]).wait()

    start_gather(0, 0)
    start_gather(1, 1)

    def outer(jj, carry):
        for b in range(NGBUF):
            j = jj * NGBUF + b
            ob = b % NBUF

            @pl.when(j + 2 < NCH)
            def _():
                start_gather(j + 2, (b + 2) % NGBUF)

            wait_gather(b)

            @pl.when(j >= NBUF)
            def _():
                wait_store(ob)

            @plsc.parallel_loop(0, C, unroll=4)
            def _row(r):
                for g in range(HPACK // LANES):
                    sl = pl.ds(g * LANES, LANES)
                    wa = bufs[b, r, sl]
                    wb = bufs[b, r + C, sl]
                    lo = (lax.bitcast_convert_type(wa << 16, jnp.float32)
                          + lax.bitcast_convert_type(wb << 16, jnp.float32))
                    hi = (lax.bitcast_convert_type(wa & MASKHI, jnp.float32)
                          + lax.bitcast_convert_type(wb & MASKHI, jnp.float32))
                    outb[ob, r, pl.ds(g * 2 * LANES, LANES)] = lo
                    outb[ob, r, pl.ds(g * 2 * LANES + LANES, LANES)] = hi

            pltpu.async_copy(outb.at[ob],
                             out_hbm.at[pl.ds(base + j * C, C)], sem_s[ob])
        return carry

    lax.fori_loop(0, NCH // NGBUF, outer, 0)
    wait_store(0)
    wait_store(1)


_emb = functools.partial(
    pl.kernel,
    mesh=plsc.VectorSubcoreMesh(core_axis_name="c", subcore_axis_name="s"),
    out_type=jax.ShapeDtypeStruct((N, HIDDEN), jnp.float32),
    scratch_types=[
        pltpu.VMEM((NCH, 2 * C), jnp.int32),
        pltpu.VMEM((NGBUF, 2 * C, HPACK), jnp.int32),
        pltpu.VMEM((NBUF, C, HIDDEN), jnp.float32),
        pltpu.SemaphoreType.DMA,
        pltpu.SemaphoreType.DMA,
        pltpu.SemaphoreType.DMA,
        pltpu.SemaphoreType.DMA,
        pltpu.SemaphoreType.DMA,
        pltpu.SemaphoreType.DMA,
    ],
)(_emb_body)


def _pack_table(W):
    v = W.shape[0]
    u = lax.bitcast_convert_type(W.astype(jnp.bfloat16), jnp.uint16)
    u = u.astype(jnp.uint32).reshape(v, HIDDEN // 32, 2, LANES)
    words = u[:, :, 0, :] | (u[:, :, 1, :] << 16)
    return lax.bitcast_convert_type(words, jnp.int32).reshape(v, HPACK)


def kernel(token_type_ids, W1, W2):
    ids = token_type_ids.astype(jnp.int32)
    idx1 = ids[:, :, 1].reshape(NW, NCH, C)
    idx2 = ids[:, :, 2].reshape(NW, NCH, C) + 256
    idxc = jnp.concatenate([idx1, idx2], axis=2)
    wp = jnp.concatenate([_pack_table(W1), _pack_table(W2)], axis=0)
    out = _emb(idxc, wp)
    return out.reshape(B, S, HIDDEN)
